# baseline (device time: 189301 ns/iter reference)
import jax
import jax.numpy as jnp
from jax import lax
from jax.experimental import pallas as pl
from jax.experimental.pallas import tpu as pltpu

N_DEV = 32
NP = 8
NZ = 4

_RING = [0, 1, 2, 5, 6, 7, 4, 3]
_R_OF = [0, 1, 2, 7, 6, 3, 4, 5]
_RIGHT_M = [_RING[(_R_OF[k] + 1) % NP] for k in range(NP)]
_LEFT_M = [_RING[(_R_OF[k] - 1) % NP] for k in range(NP)]


def _lut(idx, table):
    out = jnp.int32(0)
    for k, v in enumerate(table):
        out = out + jnp.where(idx == k, jnp.int32(v), jnp.int32(0))
    return out


class _Fam:

    def __init__(self, base, rows1, K1, pos1, dev1, rows2, K2, pos2, dev2,
                 sgn, sb1, sb2, rc1, rc2, ss, rs1, rs2, ag2, ag1):
        self.base, self.sgn = base, sgn
        self.rows1, self.K1, self.pos1, self.dev1 = rows1, K1, pos1, dev1
        self.rows2, self.K2, self.pos2, self.dev2 = rows2, K2, pos2, dev2
        self.sb1, self.sb2, self.rc1, self.rc2 = sb1, sb2, rc1, rc2
        self.ss, self.rs1, self.rs2, self.ag2, self.ag1 = ss, rs1, rs2, ag2, ag1
        self.row1 = None
        self.row2 = None


def kernel(A, B):
    m_rows, k = A.shape
    _, n = B.shape
    assert m_rows == 1536

    def body(a_ref, b_ref, out_ref, p_ref, *scr):
        my = lax.axis_index("i")
        q = my // NP
        m = my % NP
        r = _lut(m, _R_OF)
        right_dev = q * NP + _lut(m, _RIGHT_M)
        left_dev = q * NP + _lut(m, _LEFT_M)
        up_dev = ((q + 1) % NZ) * NP + m
        down_dev = ((q - 1) % NZ) * NP + m

        barrier_sem = pltpu.get_barrier_semaphore()
        for nbr in (left_dev, right_dev, up_dev, down_dev):
            pl.semaphore_signal(
                barrier_sem, inc=1,
                device_id=(nbr,), device_id_type=pl.DeviceIdType.MESH,
            )
        pl.semaphore_wait(barrier_sem, 4)

        bufs = scr[:16]
        sems = scr[16:]
        fams = []
        cfg = [
            (0,    32, NP, r, right_dev, 8,  NZ, q, up_dev,   +1),
            (256,  32, NP, r, left_dev,  8,  NZ, q, down_dev, -1),
            (512, 128, NZ, q, up_dev,   16,  NP, r, right_dev, +1),
            (1024, 128, NZ, q, down_dev, 16, NP, r, left_dev,  -1),
        ]
        for i, (base, r1, K1, p1, d1, r2, K2, p2, d2, sg) in enumerate(cfg):
            fams.append(_Fam(
                base, r1, K1, p1, d1, r2, K2, p2, d2, sg,
                bufs[4 * i], bufs[4 * i + 1], bufs[4 * i + 2], bufs[4 * i + 3],
                sems[5 * i], sems[5 * i + 1], sems[5 * i + 2],
                sems[5 * i + 3], sems[5 * i + 4],
            ))
        pfc, pfx, zfu, zfd = fams

        def mm(row0, nrows):
            p_ref[pl.ds(row0, nrows), :] = jnp.dot(
                a_ref[pl.ds(row0, nrows), :], b_ref[...],
                preferred_element_type=jnp.float32,
            )

        def copy(src, dst, ssem, rsem, dev):
            rd = pltpu.make_async_remote_copy(
                src_ref=src, dst_ref=dst, send_sem=ssem, recv_sem=rsem,
                device_id=(dev,), device_id_type=pl.DeviceIdType.MESH,
            )
            rd.start()
            return rd

        def rs1_step(f, s):
            idx = (f.pos1 - f.sgn * s) % f.K1
            val = p_ref[pl.ds(f.base + idx * f.rows1, f.rows1), :]
            if s > 0:
                val = val + f.rc1[s - 1]
            f.sb1[s % 2, :, :] = val
            return copy(f.sb1.at[s % 2], f.rc1.at[s],
                        f.ss.at[s % 2], f.rs1.at[s], f.dev1)

        def rs1_end(f):
            own1 = (f.pos1 + f.sgn) % f.K1
            f.row1 = f.base + own1 * f.rows1
            sl = pl.ds(f.row1, f.rows1)
            p_ref[sl, :] = p_ref[sl, :] + f.rc1[f.K1 - 2]

        def rs2_step(f, s):
            j = (f.pos2 - f.sgn * s) % f.K2
            val = p_ref[pl.ds(f.row1 + j * f.rows2, f.rows2), :]
            if s > 0:
                val = val + f.rc2[s - 1]
            f.sb2[s % 2, :, :] = val
            return copy(f.sb2.at[s % 2], f.rc2.at[s],
                        f.ss.at[s % 2], f.rs2.at[s], f.dev2)

        def rs2_end(f):
            own2 = (f.pos2 + f.sgn) % f.K2
            f.row2 = f.row1 + own2 * f.rows2
            sl = pl.ds(f.row2, f.rows2)
            out_ref[sl, :] = jnp.maximum(p_ref[sl, :] + f.rc2[f.K2 - 2], 0.0)

        def ag2_step(f, s):
            j = (f.pos2 + f.sgn - f.sgn * s) % f.K2
            sl = pl.ds(f.row1 + j * f.rows2, f.rows2)
            return copy(out_ref.at[sl, :], out_ref.at[sl, :],
                        f.ss.at[s % 2], f.ag2.at[s], f.dev2)

        def ag1_step(f, s):
            c = (f.pos1 + f.sgn - f.sgn * s) % f.K1
            sl = pl.ds(f.base + c * f.rows1, f.rows1)
            return copy(out_ref.at[sl, :], out_ref.at[sl, :],
                        f.ss.at[s % 2], f.ag1.at[s], f.dev1)

        mm(pfc.base + r * 32, 32)
        mm(pfx.base + r * 32, 32)
        mm(zfu.base + q * 128, 128)
        mm(zfd.base + q * 128, 128)

        for t in range(10):
            rds = []
            if t < 7:
                rds += [rs1_step(pfc, t), rs1_step(pfx, t)]
            else:
                if t == 7:
                    rs1_end(pfc)
                    rs1_end(pfx)
                rds += [rs2_step(pfc, t - 7), rs2_step(pfx, t - 7)]
            if t < 3:
                rds += [rs1_step(zfu, t), rs1_step(zfd, t)]
            else:
                if t == 3:
                    rs1_end(zfu)
                    rs1_end(zfd)
                rds += [rs2_step(zfu, t - 3), rs2_step(zfd, t - 3)]
            if t < 7:
                mm(pfc.base + ((r - t - 1) % NP) * 32, 32)
                mm(pfx.base + ((r + t + 1) % NP) * 32, 32)
            if t < 3:
                mm(zfu.base + ((q - t - 1) % NZ) * 128, 128)
                mm(zfd.base + ((q + t + 1) % NZ) * 128, 128)
            for rd in rds:
                rd.wait_send()
            for rd in rds:
                rd.wait_recv()

        for f in fams:
            rs2_end(f)

        for t in range(10):
            rds = []
            if t < 3:
                rds += [ag2_step(pfc, t), ag2_step(pfx, t)]
            else:
                rds += [ag1_step(pfc, t - 3), ag1_step(pfx, t - 3)]
            if t < 7:
                rds += [ag2_step(zfu, t), ag2_step(zfd, t)]
            else:
                rds += [ag1_step(zfu, t - 7), ag1_step(zfd, t - 7)]
            for rd in rds:
                rd.wait_send()
            for rd in rds:
                rd.wait_recv()

    scratch = [pltpu.VMEM((m_rows, n), jnp.float32)]
    fam_shapes = [
        (32, NP, 8, NZ),
        (32, NP, 8, NZ),
        (128, NZ, 16, NP),
        (128, NZ, 16, NP),
    ]
    for r1, K1, r2, K2 in fam_shapes:
        scratch += [
            pltpu.VMEM((2, r1, n), jnp.float32),
            pltpu.VMEM((2, r2, n), jnp.float32),
            pltpu.VMEM((K1 - 1, r1, n), jnp.float32),
            pltpu.VMEM((K2 - 1, r2, n), jnp.float32),
        ]
    for r1, K1, r2, K2 in fam_shapes:
        scratch += [
            pltpu.SemaphoreType.DMA((2,)),
            pltpu.SemaphoreType.DMA((K1 - 1,)),
            pltpu.SemaphoreType.DMA((K2 - 1,)),
            pltpu.SemaphoreType.DMA((K2 - 1,)),
            pltpu.SemaphoreType.DMA((K1 - 1,)),
        ]

    return pl.pallas_call(
        body,
        out_shape=jax.ShapeDtypeStruct((m_rows, n), jnp.float32),
        in_specs=[
            pl.BlockSpec(memory_space=pltpu.VMEM),
            pl.BlockSpec(memory_space=pltpu.VMEM),
        ],
        out_specs=pl.BlockSpec(memory_space=pltpu.VMEM),
        scratch_shapes=scratch,
        compiler_params=pltpu.CompilerParams(collective_id=0),
    )(A, B)


# device time: 159103 ns/iter; 1.1898x vs baseline; 1.1898x over previous
import jax
import jax.numpy as jnp
from jax import lax
from jax.experimental import pallas as pl
from jax.experimental.pallas import tpu as pltpu

N_DEV = 32
NP = 8
NZ = 4

_RING = [0, 1, 2, 5, 6, 7, 4, 3]
_R_OF = [0, 1, 2, 7, 6, 3, 4, 5]
_RIGHT_M = [_RING[(_R_OF[k] + 1) % NP] for k in range(NP)]
_LEFT_M = [_RING[(_R_OF[k] - 1) % NP] for k in range(NP)]


def _lut(idx, table):
    out = jnp.int32(0)
    for k, v in enumerate(table):
        out = out + jnp.where(idx == k, jnp.int32(v), jnp.int32(0))
    return out


def kernel(A, B):
    m_rows, k = A.shape
    _, n = B.shape
    BLK = m_rows // NP
    HALF = BLK // 2
    SLIV = HALF // NZ

    def body(a_ref, b_ref, out_ref, p_ref,
             sbuf_cw, sbuf_ccw, rs_recv_cw, rs_recv_ccw,
             zbuf_cw, zbuf_ccw, zrecv_cw, zrecv_ccw,
             ssem_cw, ssem_ccw, zs_cw, zs_ccw,
             rsem_cw, rsem_ccw, zr_cw, zr_ccw,
             agz_cw, agz_ccw, aga_cw, aga_ccw):
        my = lax.axis_index("i")
        q = my // NP
        m = my % NP
        r = _lut(m, _R_OF)
        right_dev = q * NP + _lut(m, _RIGHT_M)
        left_dev = q * NP + _lut(m, _LEFT_M)
        up_dev = ((q + 1) % NZ) * NP + m
        down_dev = ((q - 1) % NZ) * NP + m

        barrier_sem = pltpu.get_barrier_semaphore()
        for nbr in (left_dev, right_dev, up_dev, down_dev):
            pl.semaphore_signal(
                barrier_sem, inc=1,
                device_id=(nbr,), device_id_type=pl.DeviceIdType.MESH,
            )
        pl.semaphore_wait(barrier_sem, 4)

        def mm(row0, nrows):
            p_ref[pl.ds(row0, nrows), :] = jnp.dot(
                a_ref[pl.ds(row0, nrows), :], b_ref[...],
                preferred_element_type=jnp.float32,
            )

        mm(r * BLK, BLK)

        for s in range(NP - 1):
            idx_cw = (r - s) % NP
            idx_ccw = (r + s) % NP
            val_cw = p_ref[pl.ds(idx_cw * BLK, HALF), :]
            val_ccw = p_ref[pl.ds(idx_ccw * BLK + HALF, HALF), :]
            if s > 0:
                val_cw = val_cw + rs_recv_cw[s - 1]
                val_ccw = val_ccw + rs_recv_ccw[s - 1]
            sbuf_cw[s % 2, :, :] = val_cw
            sbuf_ccw[s % 2, :, :] = val_ccw
            rd_cw = pltpu.make_async_remote_copy(
                src_ref=sbuf_cw.at[s % 2], dst_ref=rs_recv_cw.at[s],
                send_sem=ssem_cw.at[s % 2], recv_sem=rsem_cw.at[s],
                device_id=(right_dev,), device_id_type=pl.DeviceIdType.MESH,
            )
            rd_ccw = pltpu.make_async_remote_copy(
                src_ref=sbuf_ccw.at[s % 2], dst_ref=rs_recv_ccw.at[s],
                send_sem=ssem_ccw.at[s % 2], recv_sem=rsem_ccw.at[s],
                device_id=(left_dev,), device_id_type=pl.DeviceIdType.MESH,
            )
            rd_cw.start()
            rd_ccw.start()
            mm(((r - s - 1) % NP) * BLK, HALF)
            mm(((r + s + 1) % NP) * BLK + HALF, HALF)
            rd_cw.wait_send()
            rd_ccw.wait_send()
            rd_cw.wait_recv()
            rd_ccw.wait_recv()

        base_cw = ((r + 1) % NP) * BLK
        base_ccw = ((r - 1) % NP) * BLK + HALF
        p_ref[pl.ds(base_cw, HALF), :] = (
            p_ref[pl.ds(base_cw, HALF), :] + rs_recv_cw[NP - 2]
        )
        p_ref[pl.ds(base_ccw, HALF), :] = (
            p_ref[pl.ds(base_ccw, HALF), :] + rs_recv_ccw[NP - 2]
        )

        for s in range(NZ - 1):
            j_cw = (q - s) % NZ
            j_ccw = (q + s) % NZ
            val_cw = p_ref[pl.ds(base_cw + j_cw * SLIV, SLIV), :]
            val_ccw = p_ref[pl.ds(base_ccw + j_ccw * SLIV, SLIV), :]
            if s > 0:
                val_cw = val_cw + zrecv_cw[s - 1]
                val_ccw = val_ccw + zrecv_ccw[s - 1]
            zbuf_cw[s % 2, :, :] = val_cw
            zbuf_ccw[s % 2, :, :] = val_ccw
            rd_cw = pltpu.make_async_remote_copy(
                src_ref=zbuf_cw.at[s % 2], dst_ref=zrecv_cw.at[s],
                send_sem=zs_cw.at[s % 2], recv_sem=zr_cw.at[s],
                device_id=(up_dev,), device_id_type=pl.DeviceIdType.MESH,
            )
            rd_ccw = pltpu.make_async_remote_copy(
                src_ref=zbuf_ccw.at[s % 2], dst_ref=zrecv_ccw.at[s],
                send_sem=zs_ccw.at[s % 2], recv_sem=zr_ccw.at[s],
                device_id=(down_dev,), device_id_type=pl.DeviceIdType.MESH,
            )
            rd_cw.start()
            rd_ccw.start()
            rd_cw.wait_send()
            rd_ccw.wait_send()
            rd_cw.wait_recv()
            rd_ccw.wait_recv()

        j_own_cw = (q + 1) % NZ
        j_own_ccw = (q - 1) % NZ
        row_cw = base_cw + j_own_cw * SLIV
        row_ccw = base_ccw + j_own_ccw * SLIV
        out_ref[pl.ds(row_cw, SLIV), :] = jnp.maximum(
            p_ref[pl.ds(row_cw, SLIV), :] + zrecv_cw[NZ - 2], 0.0
        )
        out_ref[pl.ds(row_ccw, SLIV), :] = jnp.maximum(
            p_ref[pl.ds(row_ccw, SLIV), :] + zrecv_ccw[NZ - 2], 0.0
        )

        for s in range(NZ - 1):
            j_cw = (q + 1 - s) % NZ
            j_ccw = (q - 1 + s) % NZ
            rd_cw = pltpu.make_async_remote_copy(
                src_ref=out_ref.at[pl.ds(base_cw + j_cw * SLIV, SLIV), :],
                dst_ref=out_ref.at[pl.ds(base_cw + j_cw * SLIV, SLIV), :],
                send_sem=zs_cw.at[s % 2], recv_sem=agz_cw.at[s],
                device_id=(up_dev,), device_id_type=pl.DeviceIdType.MESH,
            )
            rd_ccw = pltpu.make_async_remote_copy(
                src_ref=out_ref.at[pl.ds(base_ccw + j_ccw * SLIV, SLIV), :],
                dst_ref=out_ref.at[pl.ds(base_ccw + j_ccw * SLIV, SLIV), :],
                send_sem=zs_ccw.at[s % 2], recv_sem=agz_ccw.at[s],
                device_id=(down_dev,), device_id_type=pl.DeviceIdType.MESH,
            )
            rd_cw.start()
            rd_ccw.start()
            rd_cw.wait_send()
            rd_ccw.wait_send()
            rd_cw.wait_recv()
            rd_ccw.wait_recv()

        for s in range(NP - 1):
            c_cw = (r + 1 - s) % NP
            c_ccw = (r - 1 + s) % NP
            rd_cw = pltpu.make_async_remote_copy(
                src_ref=out_ref.at[pl.ds(c_cw * BLK, HALF), :],
                dst_ref=out_ref.at[pl.ds(c_cw * BLK, HALF), :],
                send_sem=ssem_cw.at[s % 2], recv_sem=aga_cw.at[s],
                device_id=(right_dev,), device_id_type=pl.DeviceIdType.MESH,
            )
            rd_ccw = pltpu.make_async_remote_copy(
                src_ref=out_ref.at[pl.ds(c_ccw * BLK + HALF, HALF), :],
                dst_ref=out_ref.at[pl.ds(c_ccw * BLK + HALF, HALF), :],
                send_sem=ssem_ccw.at[s % 2], recv_sem=aga_ccw.at[s],
                device_id=(left_dev,), device_id_type=pl.DeviceIdType.MESH,
            )
            rd_cw.start()
            rd_ccw.start()
            rd_cw.wait_send()
            rd_ccw.wait_send()
            rd_cw.wait_recv()
            rd_ccw.wait_recv()

    return pl.pallas_call(
        body,
        out_shape=jax.ShapeDtypeStruct((m_rows, n), jnp.float32),
        in_specs=[
            pl.BlockSpec(memory_space=pltpu.VMEM),
            pl.BlockSpec(memory_space=pltpu.VMEM),
        ],
        out_specs=pl.BlockSpec(memory_space=pltpu.VMEM),
        scratch_shapes=[
            pltpu.VMEM((m_rows, n), jnp.float32),
            pltpu.VMEM((2, HALF, n), jnp.float32),
            pltpu.VMEM((2, HALF, n), jnp.float32),
            pltpu.VMEM((NP - 1, HALF, n), jnp.float32),
            pltpu.VMEM((NP - 1, HALF, n), jnp.float32),
            pltpu.VMEM((2, SLIV, n), jnp.float32),
            pltpu.VMEM((2, SLIV, n), jnp.float32),
            pltpu.VMEM((NZ - 1, SLIV, n), jnp.float32),
            pltpu.VMEM((NZ - 1, SLIV, n), jnp.float32),
            pltpu.SemaphoreType.DMA((2,)),
            pltpu.SemaphoreType.DMA((2,)),
            pltpu.SemaphoreType.DMA((2,)),
            pltpu.SemaphoreType.DMA((2,)),
            pltpu.SemaphoreType.DMA((NP - 1,)),
            pltpu.SemaphoreType.DMA((NP - 1,)),
            pltpu.SemaphoreType.DMA((NZ - 1,)),
            pltpu.SemaphoreType.DMA((NZ - 1,)),
            pltpu.SemaphoreType.DMA((NZ - 1,)),
            pltpu.SemaphoreType.DMA((NZ - 1,)),
            pltpu.SemaphoreType.DMA((NP - 1,)),
            pltpu.SemaphoreType.DMA((NP - 1,)),
        ],
        compiler_params=pltpu.CompilerParams(collective_id=0),
    )(A, B)


# device time: 139994 ns/iter; 1.3522x vs baseline; 1.1365x over previous
import jax
import jax.numpy as jnp
from jax import lax
from jax.experimental import pallas as pl
from jax.experimental.pallas import tpu as pltpu

N_DEV = 32
NP = 8
NZ = 4

_RING = [0, 1, 2, 5, 6, 7, 4, 3]
_R_OF = [0, 1, 2, 7, 6, 3, 4, 5]
_RIGHT_M = [_RING[(_R_OF[k] + 1) % NP] for k in range(NP)]
_LEFT_M = [_RING[(_R_OF[k] - 1) % NP] for k in range(NP)]


def _lut(idx, table):
    out = jnp.int32(0)
    for k, v in enumerate(table):
        out = out + jnp.where(idx == k, jnp.int32(v), jnp.int32(0))
    return out


class _Sub:
    def __init__(self, sgn, off, dev, sb, rc, ss, rs, ag):
        self.sgn, self.off, self.dev = sgn, off, dev
        self.sb, self.rc, self.ss, self.rs, self.ag = sb, rc, ss, rs, ag
        self.rds = {}


def kernel(A, B):
    m_rows, k = A.shape
    _, n = B.shape
    BLK = m_rows // NP
    HALF = BLK // 2
    SUB = HALF // 2
    SLIV = HALF // NZ

    def body(a_ref, b_ref, out_ref, p_ref,
             sb_cwa, sb_cwb, sb_xa, sb_xb,
             rc_cwa, rc_cwb, rc_xa, rc_xb,
             zbuf_cw, zbuf_ccw, zrecv_cw, zrecv_ccw,
             ss_cwa, ss_cwb, ss_xa, ss_xb,
             rs_cwa, rs_cwb, rs_xa, rs_xb,
             zs_cw, zs_ccw, zr_cw, zr_ccw, agz_cw, agz_ccw,
             ag_cwa, ag_cwb, ag_xa, ag_xb):
        my = lax.axis_index("i")
        q = my // NP
        m = my % NP
        r = _lut(m, _R_OF)
        right_dev = q * NP + _lut(m, _RIGHT_M)
        left_dev = q * NP + _lut(m, _LEFT_M)
        up_dev = ((q + 1) % NZ) * NP + m
        down_dev = ((q - 1) % NZ) * NP + m

        barrier_sem = pltpu.get_barrier_semaphore()
        for nbr in (left_dev, right_dev, up_dev, down_dev):
            pl.semaphore_signal(
                barrier_sem, inc=1,
                device_id=(nbr,), device_id_type=pl.DeviceIdType.MESH,
            )
        pl.semaphore_wait(barrier_sem, 4)

        subs = [
            _Sub(+1, 0, right_dev, sb_cwa, rc_cwa, ss_cwa, rs_cwa, ag_cwa),
            _Sub(+1, SUB, right_dev, sb_cwb, rc_cwb, ss_cwb, rs_cwb, ag_cwb),
            _Sub(-1, HALF, left_dev, sb_xa, rc_xa, ss_xa, rs_xa, ag_xa),
            _Sub(-1, HALF + SUB, left_dev, sb_xb, rc_xb, ss_xb, rs_xb, ag_xb),
        ]

        def mm(row0, nrows):
            p_ref[pl.ds(row0, nrows), :] = jnp.dot(
                a_ref[pl.ds(row0, nrows), :], b_ref[...],
                preferred_element_type=jnp.float32,
            )

        def copy(src, dst, ssem, rsem, dev):
            rd = pltpu.make_async_remote_copy(
                src_ref=src, dst_ref=dst, send_sem=ssem, recv_sem=rsem,
                device_id=(dev,), device_id_type=pl.DeviceIdType.MESH,
            )
            rd.start()
            return rd

        def rs_issue(sub, s):
            idx = (r - sub.sgn * s) % NP
            val = p_ref[pl.ds(idx * BLK + sub.off, SUB), :]
            if s > 0:
                val = val + sub.rc[s - 1]
            sub.sb[s % 2, :, :] = val
            sub.rds[s] = copy(sub.sb.at[s % 2], sub.rc.at[s],
                              sub.ss.at[s % 2], sub.rs.at[s], sub.dev)

        mm(r * BLK, BLK)
        for sub in subs:
            rs_issue(sub, 0)
        mm(((r - 1) % NP) * BLK, HALF)
        mm(((r + 1) % NP) * BLK + HALF, HALF)
        for s in range(NP - 1):
            for sub in subs:
                sub.rds[s].wait_recv()
                if s >= 1:
                    sub.rds[s - 1].wait_send()
                if s < NP - 2:
                    rs_issue(sub, s + 1)
            if s < NP - 2:
                mm(((r - s - 2) % NP) * BLK, HALF)
                mm(((r + s + 2) % NP) * BLK + HALF, HALF)
        for sub in subs:
            sub.rds[NP - 2].wait_send()

        base_cw = ((r + 1) % NP) * BLK
        base_ccw = ((r - 1) % NP) * BLK + HALF
        for sub in subs:
            own_row = ((r + sub.sgn) % NP) * BLK + sub.off
            sl = pl.ds(own_row, SUB)
            p_ref[sl, :] = p_ref[sl, :] + sub.rc[NP - 2]

        for s in range(NZ - 1):
            j_cw = (q - s) % NZ
            j_ccw = (q + s) % NZ
            val_cw = p_ref[pl.ds(base_cw + j_cw * SLIV, SLIV), :]
            val_ccw = p_ref[pl.ds(base_ccw + j_ccw * SLIV, SLIV), :]
            if s > 0:
                val_cw = val_cw + zrecv_cw[s - 1]
                val_ccw = val_ccw + zrecv_ccw[s - 1]
            zbuf_cw[s % 2, :, :] = val_cw
            zbuf_ccw[s % 2, :, :] = val_ccw
            rd_cw = copy(zbuf_cw.at[s % 2], zrecv_cw.at[s],
                         zs_cw.at[s % 2], zr_cw.at[s], up_dev)
            rd_ccw = copy(zbuf_ccw.at[s % 2], zrecv_ccw.at[s],
                          zs_ccw.at[s % 2], zr_ccw.at[s], down_dev)
            rd_cw.wait_send()
            rd_ccw.wait_send()
            rd_cw.wait_recv()
            rd_ccw.wait_recv()

        row_cw = base_cw + ((q + 1) % NZ) * SLIV
        row_ccw = base_ccw + ((q - 1) % NZ) * SLIV
        out_ref[pl.ds(row_cw, SLIV), :] = jnp.maximum(
            p_ref[pl.ds(row_cw, SLIV), :] + zrecv_cw[NZ - 2], 0.0
        )
        out_ref[pl.ds(row_ccw, SLIV), :] = jnp.maximum(
            p_ref[pl.ds(row_ccw, SLIV), :] + zrecv_ccw[NZ - 2], 0.0
        )

        for s in range(NZ - 1):
            j_cw = (q + 1 - s) % NZ
            j_ccw = (q - 1 + s) % NZ
            sl_cw = pl.ds(base_cw + j_cw * SLIV, SLIV)
            sl_ccw = pl.ds(base_ccw + j_ccw * SLIV, SLIV)
            rd_cw = copy(out_ref.at[sl_cw, :], out_ref.at[sl_cw, :],
                         zs_cw.at[s % 2], agz_cw.at[s], up_dev)
            rd_ccw = copy(out_ref.at[sl_ccw, :], out_ref.at[sl_ccw, :],
                          zs_ccw.at[s % 2], agz_ccw.at[s], down_dev)
            rd_cw.wait_send()
            rd_ccw.wait_send()
            rd_cw.wait_recv()
            rd_ccw.wait_recv()

        def ag_issue(sub, s):
            c = (r + sub.sgn - sub.sgn * s) % NP
            sl = pl.ds(c * BLK + sub.off, SUB)
            sub.rds[s] = copy(out_ref.at[sl, :], out_ref.at[sl, :],
                              sub.ss.at[s % 2], sub.ag.at[s], sub.dev)

        for sub in subs:
            ag_issue(sub, 0)
        for s in range(NP - 1):
            for sub in subs:
                sub.rds[s].wait_recv()
                if s >= 1:
                    sub.rds[s - 1].wait_send()
                if s < NP - 2:
                    ag_issue(sub, s + 1)
        for sub in subs:
            sub.rds[NP - 2].wait_send()

    scratch = [pltpu.VMEM((m_rows, n), jnp.float32)]
    scratch += [pltpu.VMEM((2, SUB, n), jnp.float32)] * 4
    scratch += [pltpu.VMEM((NP - 1, SUB, n), jnp.float32)] * 4
    scratch += [pltpu.VMEM((2, SLIV, n), jnp.float32)] * 2
    scratch += [pltpu.VMEM((NZ - 1, SLIV, n), jnp.float32)] * 2
    scratch += [pltpu.SemaphoreType.DMA((2,))] * 4
    scratch += [pltpu.SemaphoreType.DMA((NP - 1,))] * 4
    scratch += [pltpu.SemaphoreType.DMA((2,))] * 2
    scratch += [pltpu.SemaphoreType.DMA((NZ - 1,))] * 2
    scratch += [pltpu.SemaphoreType.DMA((NZ - 1,))] * 2
    scratch += [pltpu.SemaphoreType.DMA((NP - 1,))] * 4

    return pl.pallas_call(
        body,
        out_shape=jax.ShapeDtypeStruct((m_rows, n), jnp.float32),
        in_specs=[
            pl.BlockSpec(memory_space=pltpu.VMEM),
            pl.BlockSpec(memory_space=pltpu.VMEM),
        ],
        out_specs=pl.BlockSpec(memory_space=pltpu.VMEM),
        scratch_shapes=scratch,
        compiler_params=pltpu.CompilerParams(collective_id=0),
    )(A, B)
